# baseline (speedup 1.0000x reference)
import jax
import jax.numpy as jnp
from jax.experimental import pallas as pl


_COO = ((0, 1.0), (5, 2.0), (10, 3.0))


def _add_body(x_ref, o_ref):
    lane = jax.lax.broadcasted_iota(jnp.int32, x_ref.shape, 1)
    r = lane % 16
    c = jnp.zeros(x_ref.shape, jnp.float32)
    for idx, val in _COO:
        c = c + jnp.where(r == idx, jnp.float32(val), jnp.float32(0.0))
    o_ref[...] = x_ref[...] + c


def kernel(x):
    n = x.shape[0]
    x2 = x.reshape(n // 64, 1024)
    rows = x2.shape[0]
    br = min(1024, rows)
    out = pl.pallas_call(
        _add_body,
        grid=(rows // br,),
        in_specs=[pl.BlockSpec((br, 1024), lambda i: (i, 0))],
        out_specs=pl.BlockSpec((br, 1024), lambda i: (i, 0)),
        out_shape=jax.ShapeDtypeStruct((rows, 1024), x.dtype),
    )(x2)
    return out.reshape(x.shape)


# speedup vs baseline: 239.2862x; 239.2862x over previous
import jax
import jax.numpy as jnp
from jax.experimental import pallas as pl


_COO = ((0, 0, 1.0), (1, 1, 2.0), (2, 2, 3.0))
_BC = 65536


def _add_body(x_ref, o_ref):
    j = jax.lax.broadcasted_iota(jnp.int32, x_ref.shape, 0)
    k = jax.lax.broadcasted_iota(jnp.int32, x_ref.shape, 1)
    c = jnp.zeros(x_ref.shape, jnp.float32)
    for r, cc, val in _COO:
        c = c + jnp.where((j == r) & (k == cc), jnp.float32(val), jnp.float32(0.0))
    o_ref[...] = x_ref[...] + c


def kernel(x):
    n = x.shape[0]
    xt = x.transpose(1, 2, 0)
    bc = min(_BC, n)
    out_t = pl.pallas_call(
        _add_body,
        grid=(n // bc,),
        in_specs=[pl.BlockSpec((4, 4, bc), lambda i: (0, 0, i))],
        out_specs=pl.BlockSpec((4, 4, bc), lambda i: (0, 0, i)),
        out_shape=jax.ShapeDtypeStruct((4, 4, n), x.dtype),
    )(xt)
    return out_t.transpose(2, 0, 1)
